# Initial kernel scaffold; baseline (speedup 1.0000x reference)
import jax
import jax.numpy as jnp
from jax.experimental import pallas as pl


def kernel(feature_map, xyz_hr, Wq, bq, Wk, bk, Wv, bv, in_proj_w, in_proj_b, out_proj_w, out_proj_b):
    raise NotImplementedError("write your pallas kernel here")


# speedup vs baseline: 22.5132x; 22.5132x over previous
import functools

import jax
import jax.numpy as jnp
from jax import lax
from jax.experimental import pallas as pl
from jax.experimental.pallas import tpu as pltpu
from jax.experimental.pallas import tpu_sc as plsc

C = 128
NH = 8
DH = C // NH
TOPK = 5
ND, NHH, NWW = 16, 32, 32
LHW = NHH * NWW
NB = 2
KP = 8192
P = NB * KP
DHW = ND * LHW


def _sim_body(sf_ref, out_ref):
    sf = sf_ref[0, 0]
    sim = lax.dot_general(sf, sf, (((0,), (0,)), ((), ())),
                          preferred_element_type=jnp.float32)
    icol = lax.broadcasted_iota(jnp.int32, (LHW, LHW), 1)
    irow = lax.broadcasted_iota(jnp.int32, (LHW, 1), 0)
    idxs = []
    for _ in range(TOPK):
        m = jnp.max(sim, axis=1, keepdims=True)
        cand = jnp.where(sim == m, icol, LHW)
        idx = jnp.min(cand, axis=1, keepdims=True)
        sim = jnp.where(icol == idx, -jnp.inf, sim)
        idxs.append(idx)
    ws = []
    for k in range(TOPK):
        dist = jnp.abs(idxs[k] - irow).astype(jnp.float32) + 1e-05
        ws.append(1.0 / dist)
    wsum = ws[0] + ws[1] + ws[2] + ws[3] + ws[4]
    wmat = jnp.zeros((LHW, LHW), jnp.float32)
    for k in range(TOPK):
        wmat = jnp.where(icol == idxs[k], ws[k] / wsum, wmat)
    out_ref[0, 0] = lax.dot_general(wmat, sf, (((1,), (1,)), ((), ())),
                                    preferred_element_type=jnp.float32)


def _similar_table(feature_map):
    slices = jnp.transpose(feature_map, (0, 2, 1, 3, 4)).reshape(NB, ND, C, LHW)
    out = pl.pallas_call(
        _sim_body,
        grid=(NB, ND),
        in_specs=[pl.BlockSpec((1, 1, C, LHW), lambda n, d: (n, d, 0, 0))],
        out_specs=pl.BlockSpec((1, 1, LHW, C), lambda n, d: (d, n, 0, 0)),
        out_shape=jax.ShapeDtypeStruct((ND, NB, LHW, C), jnp.float32),
    )(slices)
    weighted = jnp.transpose(out, (0, 1, 3, 2))
    sim_vol = weighted.reshape(NB, C, ND, LHW)
    return jnp.transpose(sim_vol, (0, 2, 3, 1)).reshape(NB * DHW, C)


def _prep_body(xd_ref, xh_ref, xw_ref,
               bidx_ref, bw_ref, nidx_ref, nw_ref):
    xd = xd_ref[...]
    xh = xh_ref[...]
    xw = xw_ref[...]
    noff = lax.broadcasted_iota(jnp.int32, (NB, KP), 0) * DHW

    ix = ((xw + 1.0) * NWW - 1.0) / 2.0
    iy = ((xh + 1.0) * NHH - 1.0) / 2.0
    iz = ((xd + 1.0) * ND - 1.0) / 2.0
    x0 = jnp.floor(ix); y0 = jnp.floor(iy); z0 = jnp.floor(iz)
    wx1 = ix - x0; wy1 = iy - y0; wz1 = iz - z0
    wx0 = 1.0 - wx1; wy0 = 1.0 - wy1; wz0 = 1.0 - wz1
    x0i = x0.astype(jnp.int32); y0i = y0.astype(jnp.int32); z0i = z0.astype(jnp.int32)
    corner = 0
    for dz, wz in ((0, wz0), (1, wz1)):
        for dy, wy in ((0, wy0), (1, wy1)):
            for dx, wx in ((0, wx0), (1, wx1)):
                zi = z0i + dz; yi = y0i + dy; xi = x0i + dx
                mask = ((zi >= 0) & (zi < ND) & (yi >= 0) & (yi < NHH)
                        & (xi >= 0) & (xi < NWW))
                zc = jnp.clip(zi, 0, ND - 1)
                yc = jnp.clip(yi, 0, NHH - 1)
                xc = jnp.clip(xi, 0, NWW - 1)
                flat = (zc * NHH + yc) * NWW + xc + noff
                bidx_ref[corner] = flat
                bw_ref[corner] = (wz * wy * wx) * mask.astype(jnp.float32)
                corner += 1

    gd = jnp.floor((xd + 1.0) / 2.0 * (ND - 1.0))
    gh = jnp.floor((xh + 1.0) / 2.0 * (NHH - 1.0))
    gw = jnp.floor((xw + 1.0) / 2.0 * (NWW - 1.0))
    ncn_d = gd / (ND - 1) * 2 - 1
    rws = []
    masks = []
    flats = []
    for i in range(3):
        for j in range(3):
            dh = (i - 1) * (2.0 / NHH)
            dv = (j - 1) * (2.0 / NWW)
            ncn_h = (gh + dh) / (NHH - 1) * 2 - 1
            ncn_w = (gw + dv) / (NWW - 1) * 2 - 1
            sx = ((ncn_w + 1.0) * NWW - 1.0) / 2.0
            sy = ((ncn_h + 1.0) * NHH - 1.0) / 2.0
            sz = ((ncn_d + 1.0) * ND - 1.0) / 2.0
            xi = jnp.round(sx).astype(jnp.int32)
            yi = jnp.round(sy).astype(jnp.int32)
            zi = jnp.round(sz).astype(jnp.int32)
            mask = ((zi >= 0) & (zi < ND) & (yi >= 0) & (yi < NHH)
                    & (xi >= 0) & (xi < NWW))
            zc = jnp.clip(zi, 0, ND - 1)
            yc = jnp.clip(yi, 0, NHH - 1)
            xc = jnp.clip(xi, 0, NWW - 1)
            flat = (zc * NHH + yc) * NWW + xc + noff
            flats.append(jnp.where(mask, flat, NB * DHW))
            masks.append(mask)
            rd = jnp.sqrt((xd - ncn_d) ** 2 + (xh - ncn_h) ** 2
                          + (xw - ncn_w) ** 2)
            rws.append(1.0 / (rd + 1e-06))
    rwsum = rws[0]
    for a in range(1, 9):
        rwsum = rwsum + rws[a]
    for a in range(9):
        nidx_ref[a] = flats[a]
        nw_ref[a] = rws[a] / rwsum


def _prep(xyz_hr):
    xd = xyz_hr[:, :, 0]
    xh = xyz_hr[:, :, 1]
    xw = xyz_hr[:, :, 2]
    outs = pl.pallas_call(
        _prep_body,
        in_specs=[pl.BlockSpec((NB, KP), lambda: (0, 0))] * 3,
        out_specs=[
            pl.BlockSpec((8, NB, KP), lambda: (0, 0, 0)),
            pl.BlockSpec((8, NB, KP), lambda: (0, 0, 0)),
            pl.BlockSpec((9, NB, KP), lambda: (0, 0, 0)),
            pl.BlockSpec((9, NB, KP), lambda: (0, 0, 0)),
        ],
        out_shape=[
            jax.ShapeDtypeStruct((8, NB, KP), jnp.int32),
            jax.ShapeDtypeStruct((8, NB, KP), jnp.float32),
            jax.ShapeDtypeStruct((9, NB, KP), jnp.int32),
            jax.ShapeDtypeStruct((9, NB, KP), jnp.float32),
        ],
    )(xd, xh, xw)
    bidx, bw, nidx, nw = outs
    bidx = jnp.transpose(bidx, (1, 2, 0)).reshape(P * 8)
    bw = jnp.transpose(bw, (1, 2, 0)).reshape(P * 8)
    nidx = jnp.transpose(nidx, (1, 2, 0)).reshape(P * 9)
    rw_flat = jnp.transpose(nw, (1, 2, 0)).reshape(NB, KP * 9)
    rw_t = (jnp.transpose(rw_flat.reshape(NB, C, 576), (0, 2, 1)) * 0.5
            ).reshape(NB * 576 * C)
    return bidx, bw, nidx, rw_t


NC_SC = 2
NS_SC = 16
NWK = NC_SC * NS_SC
PTS_W = P // NWK
G = 8
BITERS = PTS_W // G
NPB = 576
UNITS = NB * 64
UNITS_W = UNITS // NWK
NCH = C // 16


def _gather_stage(fm_table, sim_table, bidx, bw, nidx, rw_t):
    mesh = plsc.VectorSubcoreMesh(core_axis_name="c", subcore_axis_name="s")

    @functools.partial(
        pl.kernel, mesh=mesh,
        out_type=[
            jax.ShapeDtypeStruct((P, C), jnp.float32),
            jax.ShapeDtypeStruct((UNITS * C, C), jnp.float32),
        ],
        scratch_types=[
            pltpu.VMEM((G * 8,), jnp.int32),
            pltpu.VMEM((G * 8,), jnp.float32),
            pltpu.VMEM((G * 8, C), jnp.float32),
            pltpu.VMEM((G, C), jnp.float32),
            pltpu.VMEM((C,), jnp.int32),
            pltpu.VMEM((C,), jnp.float32),
            pltpu.VMEM((C, C), jnp.float32),
            pltpu.VMEM((C, C), jnp.float32),
            pltpu.VMEM((C, C), jnp.float32),
            pltpu.SemaphoreType.DMA,
        ],
    )
    def sc_kernel(fm_hbm, sim_hbm, bidx_hbm, bw_hbm, nidx_hbm, rw_hbm,
                  init_hbm, u_hbm,
                  bidx_v, bw_v, brows, out_i,
                  nidx_v, wrow_v, rows_fm, rows_sim, u_acc, sem):
        wid = lax.axis_index("s") * NC_SC + lax.axis_index("c")

        def unit_body(t, _):
            unit = wid * UNITS_W + t
            nn = unit // 64
            qq = unit - nn * 64

            def zero_row(r, _z):
                for c8 in range(NCH):
                    u_acc[r, pl.ds(c8 * 16, 16)] = jnp.zeros((16,), jnp.float32)
                return _z
            lax.fori_loop(0, C, zero_row, 0)

            def sub_body(a, _s):
                sbase = pl.multiple_of(nn * (KP * 9) + qq * 1152 + a * C, C)
                wbase = pl.multiple_of((nn * NPB + qq * 9 + a) * C, C)
                pltpu.sync_copy(nidx_hbm.at[pl.ds(sbase, C)], nidx_v)
                pltpu.sync_copy(rw_hbm.at[pl.ds(wbase, C)], wrow_v)
                cp1 = pltpu.async_copy(fm_hbm.at[nidx_v], rows_fm, sem)
                cp2 = pltpu.async_copy(sim_hbm.at[nidx_v], rows_sim, sem)
                cp1.wait()
                cp2.wait()
                wchunks = [wrow_v[pl.ds(c8 * 16, 16)] for c8 in range(NCH)]

                def row_body(r, _r):
                    for c8 in range(NCH):
                        sl = pl.ds(c8 * 16, 16)
                        val = (rows_fm[r, sl] + rows_sim[r, sl]) * wchunks[c8]
                        u_acc[r, sl] = u_acc[r, sl] + val
                    return _r
                lax.fori_loop(0, C, row_body, 0)
                return _s
            lax.fori_loop(0, 9, sub_body, 0)
            ub = pl.multiple_of(unit * C, C)
            pltpu.sync_copy(u_acc, u_hbm.at[pl.ds(ub, C)])
            return _
        lax.fori_loop(0, UNITS_W, unit_body, 0)

        wbase_pts = wid * PTS_W

        def bil_body(it, _):
            base = wbase_pts + it * G
            ob = pl.multiple_of(base, G)
            b8 = pl.multiple_of(base * 8, G * 8)
            pltpu.sync_copy(bidx_hbm.at[pl.ds(b8, G * 8)], bidx_v)
            pltpu.sync_copy(bw_hbm.at[pl.ds(b8, G * 8)], bw_v)
            pltpu.async_copy(fm_hbm.at[bidx_v], brows, sem).wait()

            def pt_body(pair, _p):
                wv = bw_v[pl.ds(pair * 16, 16)]
                for half in range(2):
                    g = pair * 2 + half
                    wb = [wv[half * 8 + j] for j in range(8)]
                    for c8 in range(NCH):
                        sl = pl.ds(c8 * 16, 16)
                        acc = wb[0] * brows[g * 8, sl]
                        for j in range(1, 8):
                            acc = acc + wb[j] * brows[g * 8 + j, sl]
                        out_i[g, sl] = acc
                return _p
            lax.fori_loop(0, G // 2, pt_body, 0)
            pltpu.sync_copy(out_i, init_hbm.at[pl.ds(ob, G)])
            return _
        lax.fori_loop(0, BITERS, bil_body, 0)

    return sc_kernel(fm_table, sim_table, bidx, bw, nidx, rw_t)


def _attn_body(init_ref, comb_ref, wq_ref, bq_ref, wk_ref, bk_ref,
               wv_ref, bv_ref, wqi_ref, wki_ref, wvi_ref, bi_ref,
               wo_ref, bo_ref, out_ref):
    def aff(x, w_ref, b=None):
        y = lax.dot_general(x, w_ref[...], (((1,), (1,)), ((), ())),
                            preferred_element_type=jnp.float32)
        if b is not None:
            y = y + b[...]
        return y

    i_l = [init_ref[0], init_ref[1]]
    c_l = [comb_ref[0], comb_ref[1]]
    q = [aff(i_l[l], wq_ref, bq_ref) for l in range(2)]
    k = [aff(c_l[l], wk_ref, bk_ref) for l in range(2)]
    v = [aff(c_l[l], wv_ref, bv_ref) for l in range(2)]
    bi = bi_ref[...]
    qp = [aff(q[l], wqi_ref) + bi[0:1] for l in range(2)]
    kp = [aff(k[l], wki_ref) + bi[1:2] for l in range(2)]
    vp = [aff(v[l], wvi_ref) + bi[2:3] for l in range(2)]

    seg = (lax.broadcasted_iota(jnp.int32, (C, NH), 0) // DH
           == lax.broadcasted_iota(jnp.int32, (C, NH), 1)).astype(jnp.float32)
    segT = (lax.broadcasted_iota(jnp.int32, (NH, C), 0)
            == lax.broadcasted_iota(jnp.int32, (NH, C), 1) // DH).astype(jnp.float32)
    scale = 1.0 / (DH ** 0.5)

    for l in range(2):
        s0 = lax.dot_general(qp[l] * kp[0], seg, (((1,), (0,)), ((), ())),
                             preferred_element_type=jnp.float32) * scale
        s1 = lax.dot_general(qp[l] * kp[1], seg, (((1,), (0,)), ((), ())),
                             preferred_element_type=jnp.float32) * scale
        m = jnp.maximum(s0, s1)
        e0 = jnp.exp(s0 - m)
        e1 = jnp.exp(s1 - m)
        den = e0 + e1
        a0 = lax.dot_general(e0 / den, segT, (((1,), (0,)), ((), ())),
                             preferred_element_type=jnp.float32)
        a1 = lax.dot_general(e1 / den, segT, (((1,), (0,)), ((), ())),
                             preferred_element_type=jnp.float32)
        o = a0 * vp[0] + a1 * vp[1]
        out_ref[l] = aff(o, wo_ref, bo_ref) + i_l[l]


def _attention(init_fv, combined, Wq, bq, Wk, bk, Wv, bv,
               in_proj_w, in_proj_b, out_proj_w, out_proj_b):
    TD = 2048
    wqi = in_proj_w[0:C]
    wki = in_proj_w[C:2 * C]
    wvi = in_proj_w[2 * C:3 * C]
    bi = in_proj_b.reshape(3, C)
    full = pl.BlockSpec((C, C), lambda t: (0, 0))
    bias = pl.BlockSpec((1, C), lambda t: (0, 0))
    return pl.pallas_call(
        _attn_body,
        grid=(KP // TD,),
        in_specs=[
            pl.BlockSpec((NB, TD, C), lambda t: (0, t, 0)),
            pl.BlockSpec((NB, TD, C), lambda t: (0, t, 0)),
            full, bias, full, bias, full, bias,
            full, full, full, pl.BlockSpec((3, C), lambda t: (0, 0)),
            full, bias,
        ],
        out_specs=pl.BlockSpec((NB, TD, C), lambda t: (0, t, 0)),
        out_shape=jax.ShapeDtypeStruct((NB, KP, C), jnp.float32),
    )(init_fv, combined, Wq, bq.reshape(1, C), Wk, bk.reshape(1, C),
      Wv, bv.reshape(1, C), wqi, wki, wvi, bi, out_proj_w,
      out_proj_b.reshape(1, C))


def kernel(feature_map, xyz_hr, Wq, bq, Wk, bk, Wv, bv,
           in_proj_w, in_proj_b, out_proj_w, out_proj_b):
    zrow = jnp.zeros((1, C), jnp.float32)
    fm_table = jnp.concatenate(
        [jnp.transpose(feature_map, (0, 2, 3, 4, 1)).reshape(NB * DHW, C), zrow])
    sim_table = jnp.concatenate([_similar_table(feature_map), zrow])
    bidx, bw, nidx, rw_t = _prep(xyz_hr)
    init_flat, u_flat = _gather_stage(fm_table, sim_table, bidx, bw, nidx, rw_t)
    init_fv = init_flat.reshape(NB, KP, C)
    combined = jnp.transpose(u_flat.reshape(NB, 64, C, C),
                             (0, 3, 1, 2)).reshape(NB, KP, C)
    return _attention(init_fv, combined, Wq, bq, Wk, bk, Wv, bv,
                      in_proj_w, in_proj_b, out_proj_w, out_proj_b)


# speedup vs baseline: 27.0461x; 1.2013x over previous
import functools

import jax
import jax.numpy as jnp
from jax import lax
from jax.experimental import pallas as pl
from jax.experimental.pallas import tpu as pltpu
from jax.experimental.pallas import tpu_sc as plsc

C = 128
NH = 8
DH = C // NH
TOPK = 5
ND, NHH, NWW = 16, 32, 32
LHW = NHH * NWW
NB = 2
KP = 8192
P = NB * KP
DHW = ND * LHW


def _sim_body(sf_ref, out_ref):
    sf = sf_ref[0, 0]
    sim = lax.dot_general(sf, sf, (((0,), (0,)), ((), ())),
                          preferred_element_type=jnp.float32)
    icol = lax.broadcasted_iota(jnp.int32, (LHW, LHW), 1)
    irow = lax.broadcasted_iota(jnp.int32, (LHW, 1), 0)
    idxs = []
    for _ in range(TOPK):
        m = jnp.max(sim, axis=1, keepdims=True)
        cand = jnp.where(sim == m, icol, LHW)
        idx = jnp.min(cand, axis=1, keepdims=True)
        sim = jnp.where(icol == idx, -jnp.inf, sim)
        idxs.append(idx)
    ws = []
    for k in range(TOPK):
        dist = jnp.abs(idxs[k] - irow).astype(jnp.float32) + 1e-05
        ws.append(1.0 / dist)
    wsum = ws[0] + ws[1] + ws[2] + ws[3] + ws[4]
    wmat = jnp.zeros((LHW, LHW), jnp.float32)
    for k in range(TOPK):
        wmat = jnp.where(icol == idxs[k], ws[k] / wsum, wmat)
    out_ref[0, 0] = lax.dot_general(wmat, sf, (((1,), (1,)), ((), ())),
                                    preferred_element_type=jnp.float32)


def _similar_table(feature_map):
    slices = jnp.transpose(feature_map, (0, 2, 1, 3, 4)).reshape(NB, ND, C, LHW)
    out = pl.pallas_call(
        _sim_body,
        grid=(NB, ND),
        in_specs=[pl.BlockSpec((1, 1, C, LHW), lambda n, d: (n, d, 0, 0))],
        out_specs=pl.BlockSpec((1, 1, LHW, C), lambda n, d: (d, n, 0, 0)),
        out_shape=jax.ShapeDtypeStruct((ND, NB, LHW, C), jnp.float32),
    )(slices)
    weighted = jnp.transpose(out, (0, 1, 3, 2))
    sim_vol = weighted.reshape(NB, C, ND, LHW)
    return jnp.transpose(sim_vol, (0, 2, 3, 1)).reshape(NB * DHW, C)


def _prep_body(xd_ref, xh_ref, xw_ref,
               bidx_ref, bw_ref, nidx_ref, nw_ref):
    xd = xd_ref[...]
    xh = xh_ref[...]
    xw = xw_ref[...]
    noff = lax.broadcasted_iota(jnp.int32, (NB, KP), 0) * DHW

    ix = ((xw + 1.0) * NWW - 1.0) / 2.0
    iy = ((xh + 1.0) * NHH - 1.0) / 2.0
    iz = ((xd + 1.0) * ND - 1.0) / 2.0
    x0 = jnp.floor(ix); y0 = jnp.floor(iy); z0 = jnp.floor(iz)
    wx1 = ix - x0; wy1 = iy - y0; wz1 = iz - z0
    wx0 = 1.0 - wx1; wy0 = 1.0 - wy1; wz0 = 1.0 - wz1
    x0i = x0.astype(jnp.int32); y0i = y0.astype(jnp.int32); z0i = z0.astype(jnp.int32)
    corner = 0
    for dz, wz in ((0, wz0), (1, wz1)):
        for dy, wy in ((0, wy0), (1, wy1)):
            for dx, wx in ((0, wx0), (1, wx1)):
                zi = z0i + dz; yi = y0i + dy; xi = x0i + dx
                mask = ((zi >= 0) & (zi < ND) & (yi >= 0) & (yi < NHH)
                        & (xi >= 0) & (xi < NWW))
                zc = jnp.clip(zi, 0, ND - 1)
                yc = jnp.clip(yi, 0, NHH - 1)
                xc = jnp.clip(xi, 0, NWW - 1)
                flat = (zc * NHH + yc) * NWW + xc + noff
                bidx_ref[corner] = flat
                bw_ref[corner] = (wz * wy * wx) * mask.astype(jnp.float32)
                corner += 1

    gd = jnp.floor((xd + 1.0) / 2.0 * (ND - 1.0))
    gh = jnp.floor((xh + 1.0) / 2.0 * (NHH - 1.0))
    gw = jnp.floor((xw + 1.0) / 2.0 * (NWW - 1.0))
    ncn_d = gd / (ND - 1) * 2 - 1
    rws = []
    masks = []
    flats = []
    for i in range(3):
        for j in range(3):
            dh = (i - 1) * (2.0 / NHH)
            dv = (j - 1) * (2.0 / NWW)
            ncn_h = (gh + dh) / (NHH - 1) * 2 - 1
            ncn_w = (gw + dv) / (NWW - 1) * 2 - 1
            sx = ((ncn_w + 1.0) * NWW - 1.0) / 2.0
            sy = ((ncn_h + 1.0) * NHH - 1.0) / 2.0
            sz = ((ncn_d + 1.0) * ND - 1.0) / 2.0
            xi = jnp.round(sx).astype(jnp.int32)
            yi = jnp.round(sy).astype(jnp.int32)
            zi = jnp.round(sz).astype(jnp.int32)
            mask = ((zi >= 0) & (zi < ND) & (yi >= 0) & (yi < NHH)
                    & (xi >= 0) & (xi < NWW))
            zc = jnp.clip(zi, 0, ND - 1)
            yc = jnp.clip(yi, 0, NHH - 1)
            xc = jnp.clip(xi, 0, NWW - 1)
            flat = (zc * NHH + yc) * NWW + xc + noff
            flats.append(jnp.where(mask, flat, NB * DHW))
            masks.append(mask)
            rd = jnp.sqrt((xd - ncn_d) ** 2 + (xh - ncn_h) ** 2
                          + (xw - ncn_w) ** 2)
            rws.append(1.0 / (rd + 1e-06))
    rwsum = rws[0]
    for a in range(1, 9):
        rwsum = rwsum + rws[a]
    for a in range(9):
        nidx_ref[a] = flats[a]
        nw_ref[a] = rws[a] / rwsum


def _prep(xyz_hr):
    xd = xyz_hr[:, :, 0]
    xh = xyz_hr[:, :, 1]
    xw = xyz_hr[:, :, 2]
    outs = pl.pallas_call(
        _prep_body,
        in_specs=[pl.BlockSpec((NB, KP), lambda: (0, 0))] * 3,
        out_specs=[
            pl.BlockSpec((8, NB, KP), lambda: (0, 0, 0)),
            pl.BlockSpec((8, NB, KP), lambda: (0, 0, 0)),
            pl.BlockSpec((9, NB, KP), lambda: (0, 0, 0)),
            pl.BlockSpec((9, NB, KP), lambda: (0, 0, 0)),
        ],
        out_shape=[
            jax.ShapeDtypeStruct((8, NB, KP), jnp.int32),
            jax.ShapeDtypeStruct((8, NB, KP), jnp.float32),
            jax.ShapeDtypeStruct((9, NB, KP), jnp.int32),
            jax.ShapeDtypeStruct((9, NB, KP), jnp.float32),
        ],
    )(xd, xh, xw)
    bidx, bw, nidx, nw = outs
    bidx = jnp.concatenate([jnp.transpose(bidx, (1, 2, 0)).reshape(P * 8),
                            jnp.zeros((G * 8,), jnp.int32)])
    bw = jnp.concatenate([jnp.transpose(bw, (1, 2, 0)).reshape(P * 8),
                          jnp.zeros((G * 8,), jnp.float32)])
    nidx = jnp.transpose(nidx, (1, 2, 0)).reshape(P * 9)
    rw_flat = jnp.transpose(nw, (1, 2, 0)).reshape(NB, KP * 9)
    rw_t = (jnp.transpose(rw_flat.reshape(NB, C, 576), (0, 2, 1)) * 0.5
            ).reshape(NB * 576 * C)
    return bidx, bw, nidx, rw_t


NC_SC = 2
NS_SC = 16
NWK = NC_SC * NS_SC
PTS_W = P // NWK
G = 8
BITERS = PTS_W // G
NPB = 576
UNITS = NB * 64
UNITS_W = UNITS // NWK
NCH = C // 16


def _gather_stage(fm_table, sim_table, bidx, bw, nidx, rw_t):
    mesh = plsc.VectorSubcoreMesh(core_axis_name="c", subcore_axis_name="s")

    @functools.partial(
        pl.kernel, mesh=mesh,
        out_type=[
            jax.ShapeDtypeStruct((P, C), jnp.float32),
            jax.ShapeDtypeStruct((UNITS * C, C), jnp.float32),
        ],
        scratch_types=[
            pltpu.VMEM((G * 8,), jnp.int32),
            pltpu.VMEM((G * 8,), jnp.int32),
            pltpu.VMEM((G * 8,), jnp.float32),
            pltpu.VMEM((G * 8,), jnp.float32),
            pltpu.VMEM((G * 8, C), jnp.float32),
            pltpu.VMEM((G * 8, C), jnp.float32),
            pltpu.VMEM((G, C), jnp.float32),
            pltpu.VMEM((C,), jnp.int32),
            pltpu.VMEM((C,), jnp.int32),
            pltpu.VMEM((C,), jnp.float32),
            pltpu.VMEM((C,), jnp.float32),
            pltpu.VMEM((C, C), jnp.float32),
            pltpu.VMEM((C, C), jnp.float32),
            pltpu.VMEM((C, C), jnp.float32),
            pltpu.VMEM((C, C), jnp.float32),
            pltpu.VMEM((C, C), jnp.float32),
            pltpu.SemaphoreType.DMA,
            pltpu.SemaphoreType.DMA,
        ],
    )
    def sc_kernel(fm_hbm, sim_hbm, bidx_hbm, bw_hbm, nidx_hbm, rw_hbm,
                  init_hbm, u_hbm,
                  bidx0, bidx1, bw0, bw1, brows0, brows1, out_i,
                  nidx0, nidx1, wrow0, wrow1,
                  rows_fm0, rows_fm1, rows_sim0, rows_sim1, u_acc,
                  sem0, sem1):
        wid = lax.axis_index("s") * NC_SC + lax.axis_index("c")
        nidx_b = (nidx0, nidx1)
        wrow_b = (wrow0, wrow1)
        rfm_b = (rows_fm0, rows_fm1)
        rsim_b = (rows_sim0, rows_sim1)
        sem_b = (sem0, sem1)

        def fetch_sub(nn, qq, a, p):
            sbase = pl.multiple_of(nn * (KP * 9) + qq * 1152 + a * C, C)
            wbase = pl.multiple_of((nn * NPB + qq * 9 + a) * C, C)
            pltpu.sync_copy(nidx_hbm.at[pl.ds(sbase, C)], nidx_b[p])
            pltpu.sync_copy(rw_hbm.at[pl.ds(wbase, C)], wrow_b[p])
            c1 = pltpu.async_copy(fm_hbm.at[nidx_b[p]], rfm_b[p], sem_b[p])
            c2 = pltpu.async_copy(sim_hbm.at[nidx_b[p]], rsim_b[p], sem_b[p])
            return c1, c2

        def unit_body(t, _):
            unit = wid * UNITS_W + t
            nn = unit // 64
            qq = unit - nn * 64

            def zero_row(r, _z):
                for c8 in range(NCH):
                    u_acc[r, pl.ds(c8 * 16, 16)] = jnp.zeros((16,), jnp.float32)
                return _z
            lax.fori_loop(0, C // 2, lambda r, z: zero_row(2 * r, zero_row(2 * r + 1, z)), 0)

            cps = fetch_sub(nn, qq, 0, 0)
            for a in range(9):
                p = a % 2
                nxt = cps
                if a + 1 < 9:
                    cps = fetch_sub(nn, qq, a + 1, 1 - p)
                nxt[0].wait()
                nxt[1].wait()
                wchunks = [wrow_b[p][pl.ds(c8 * 16, 16)] for c8 in range(NCH)]
                rfm = rfm_b[p]
                rsim = rsim_b[p]

                def row_body(r2, _r, rfm=rfm, rsim=rsim, wchunks=wchunks):
                    for u in range(2):
                        r = r2 * 2 + u
                        for c8 in range(NCH):
                            sl = pl.ds(c8 * 16, 16)
                            val = (rfm[r, sl] + rsim[r, sl]) * wchunks[c8]
                            plsc.addupdate(u_acc.at[r, sl], val)
                    return _r
                lax.fori_loop(0, C // 2, row_body, 0)
            ub = pl.multiple_of(unit * C, C)
            pltpu.sync_copy(u_acc, u_hbm.at[pl.ds(ub, C)])
            return _
        lax.fori_loop(0, UNITS_W, unit_body, 0)

        wbase_pts = wid * PTS_W
        bidx_d = (bidx0, bidx1)
        bw_d = (bw0, bw1)
        brows_d = (brows0, brows1)

        def fetch_bil(chunk, p):
            b8 = pl.multiple_of(chunk * (G * 8), G * 8)
            pltpu.sync_copy(bidx_hbm.at[pl.ds(b8, G * 8)], bidx_d[p])
            pltpu.sync_copy(bw_hbm.at[pl.ds(b8, G * 8)], bw_d[p])
            pltpu.async_copy(fm_hbm.at[bidx_d[p]], brows_d[p], sem_b[p])

        fetch_bil(wid * BITERS, 0)

        def bil2_body(i2, _):
            for b in range(2):
                chunk = i2 * 2 + b
                base = wbase_pts + chunk * G
                ob = pl.multiple_of(base, G)
                fetch_bil(wid * BITERS + chunk + 1, 1 - b)
                pltpu.make_async_copy(fm_hbm.at[bidx_d[b]],
                                      brows_d[b], sem_b[b]).wait()
                bwv = bw_d[b]
                brr = brows_d[b]

                def pt_body(pair, _p, bwv=bwv, brr=brr):
                    wv = bwv[pl.ds(pair * 16, 16)]
                    for half in range(2):
                        g = pair * 2 + half
                        wb = [wv[half * 8 + j] for j in range(8)]
                        for c8 in range(NCH):
                            sl = pl.ds(c8 * 16, 16)
                            acc = wb[0] * brr[g * 8, sl]
                            for j in range(1, 8):
                                acc = acc + wb[j] * brr[g * 8 + j, sl]
                            out_i[g, sl] = acc
                    return _p
                lax.fori_loop(0, G // 2, pt_body, 0)
                pltpu.sync_copy(out_i, init_hbm.at[pl.ds(ob, G)])
            return _
        lax.fori_loop(0, BITERS // 2, bil2_body, 0)
        pltpu.make_async_copy(fm_hbm.at[bidx_d[0]], brows_d[0], sem_b[0]).wait()

    return sc_kernel(fm_table, sim_table, bidx, bw, nidx, rw_t)


def _attn_body(init_ref, comb_ref, wq_ref, bq_ref, wk_ref, bk_ref,
               wv_ref, bv_ref, wqi_ref, wki_ref, wvi_ref, bi_ref,
               wo_ref, bo_ref, out_ref):
    def aff(x, w_ref, b=None):
        y = lax.dot_general(x, w_ref[...], (((1,), (1,)), ((), ())),
                            preferred_element_type=jnp.float32)
        if b is not None:
            y = y + b[...]
        return y

    i_l = [init_ref[0], init_ref[1]]
    c_l = [comb_ref[0], comb_ref[1]]
    q = [aff(i_l[l], wq_ref, bq_ref) for l in range(2)]
    k = [aff(c_l[l], wk_ref, bk_ref) for l in range(2)]
    v = [aff(c_l[l], wv_ref, bv_ref) for l in range(2)]
    bi = bi_ref[...]
    qp = [aff(q[l], wqi_ref) + bi[0:1] for l in range(2)]
    kp = [aff(k[l], wki_ref) + bi[1:2] for l in range(2)]
    vp = [aff(v[l], wvi_ref) + bi[2:3] for l in range(2)]

    seg = (lax.broadcasted_iota(jnp.int32, (C, NH), 0) // DH
           == lax.broadcasted_iota(jnp.int32, (C, NH), 1)).astype(jnp.float32)
    segT = (lax.broadcasted_iota(jnp.int32, (NH, C), 0)
            == lax.broadcasted_iota(jnp.int32, (NH, C), 1) // DH).astype(jnp.float32)
    scale = 1.0 / (DH ** 0.5)

    for l in range(2):
        s0 = lax.dot_general(qp[l] * kp[0], seg, (((1,), (0,)), ((), ())),
                             preferred_element_type=jnp.float32) * scale
        s1 = lax.dot_general(qp[l] * kp[1], seg, (((1,), (0,)), ((), ())),
                             preferred_element_type=jnp.float32) * scale
        m = jnp.maximum(s0, s1)
        e0 = jnp.exp(s0 - m)
        e1 = jnp.exp(s1 - m)
        den = e0 + e1
        a0 = lax.dot_general(e0 / den, segT, (((1,), (0,)), ((), ())),
                             preferred_element_type=jnp.float32)
        a1 = lax.dot_general(e1 / den, segT, (((1,), (0,)), ((), ())),
                             preferred_element_type=jnp.float32)
        o = a0 * vp[0] + a1 * vp[1]
        out_ref[l] = aff(o, wo_ref, bo_ref) + i_l[l]


def _attention(init_fv, combined, Wq, bq, Wk, bk, Wv, bv,
               in_proj_w, in_proj_b, out_proj_w, out_proj_b):
    TD = 2048
    wqi = in_proj_w[0:C]
    wki = in_proj_w[C:2 * C]
    wvi = in_proj_w[2 * C:3 * C]
    bi = in_proj_b.reshape(3, C)
    full = pl.BlockSpec((C, C), lambda t: (0, 0))
    bias = pl.BlockSpec((1, C), lambda t: (0, 0))
    return pl.pallas_call(
        _attn_body,
        grid=(KP // TD,),
        in_specs=[
            pl.BlockSpec((NB, TD, C), lambda t: (0, t, 0)),
            pl.BlockSpec((NB, TD, C), lambda t: (0, t, 0)),
            full, bias, full, bias, full, bias,
            full, full, full, pl.BlockSpec((3, C), lambda t: (0, 0)),
            full, bias,
        ],
        out_specs=pl.BlockSpec((NB, TD, C), lambda t: (0, t, 0)),
        out_shape=jax.ShapeDtypeStruct((NB, KP, C), jnp.float32),
    )(init_fv, combined, Wq, bq.reshape(1, C), Wk, bk.reshape(1, C),
      Wv, bv.reshape(1, C), wqi, wki, wvi, bi, out_proj_w,
      out_proj_b.reshape(1, C))


def kernel(feature_map, xyz_hr, Wq, bq, Wk, bk, Wv, bv,
           in_proj_w, in_proj_b, out_proj_w, out_proj_b):
    zrow = jnp.zeros((1, C), jnp.float32)
    fm_table = jnp.concatenate(
        [jnp.transpose(feature_map, (0, 2, 3, 4, 1)).reshape(NB * DHW, C), zrow])
    sim_table = jnp.concatenate([_similar_table(feature_map), zrow])
    bidx, bw, nidx, rw_t = _prep(xyz_hr)
    init_flat, u_flat = _gather_stage(fm_table, sim_table, bidx, bw, nidx, rw_t)
    init_fv = init_flat.reshape(NB, KP, C)
    combined = jnp.transpose(u_flat.reshape(NB, 64, C, C),
                             (0, 3, 1, 2)).reshape(NB, KP, C)
    return _attention(init_fv, combined, Wq, bq, Wk, bk, Wv, bv,
                      in_proj_w, in_proj_b, out_proj_w, out_proj_b)


# speedup vs baseline: 27.4290x; 1.0142x over previous
import functools

import jax
import jax.numpy as jnp
from jax import lax
from jax.experimental import pallas as pl
from jax.experimental.pallas import tpu as pltpu
from jax.experimental.pallas import tpu_sc as plsc

C = 128
NH = 8
DH = C // NH
TOPK = 5
ND, NHH, NWW = 16, 32, 32
LHW = NHH * NWW
NB = 2
KP = 8192
P = NB * KP
DHW = ND * LHW


def _sim_body(sf_ref, out_ref):
    sf = sf_ref[0, 0]
    sim = lax.dot_general(sf, sf, (((0,), (0,)), ((), ())),
                          preferred_element_type=jnp.float32)
    icol = lax.broadcasted_iota(jnp.int32, (LHW, LHW), 1)
    irow = lax.broadcasted_iota(jnp.int32, (LHW, 1), 0)
    wmat_raw = jnp.zeros((LHW, LHW), jnp.float32)
    wraws = []
    for _ in range(TOPK):
        m = jnp.max(sim, axis=1, keepdims=True)
        cand = jnp.where(sim == m, icol, LHW)
        idx = jnp.min(cand, axis=1, keepdims=True)
        wr = 1.0 / (jnp.abs(idx - irow).astype(jnp.float32) + 1e-05)
        e = icol == idx
        sim = jnp.where(e, -jnp.inf, sim)
        wmat_raw = jnp.where(e, wr, wmat_raw)
        wraws.append(wr)
    rinv = 1.0 / (wraws[0] + wraws[1] + wraws[2] + wraws[3] + wraws[4])
    out_ref[0, 0] = lax.dot_general(wmat_raw * rinv, sf,
                                    (((1,), (1,)), ((), ())),
                                    preferred_element_type=jnp.float32)


def _similar_table(feature_map):
    slices = jnp.transpose(feature_map, (0, 2, 1, 3, 4)).reshape(NB, ND, C, LHW)
    out = pl.pallas_call(
        _sim_body,
        grid=(NB, ND),
        in_specs=[pl.BlockSpec((1, 1, C, LHW), lambda n, d: (n, d, 0, 0))],
        out_specs=pl.BlockSpec((1, 1, LHW, C), lambda n, d: (d, n, 0, 0)),
        out_shape=jax.ShapeDtypeStruct((ND, NB, LHW, C), jnp.float32),
    )(slices)
    weighted = jnp.transpose(out, (0, 1, 3, 2))
    sim_vol = weighted.reshape(NB, C, ND, LHW)
    return jnp.transpose(sim_vol, (0, 2, 3, 1)).reshape(NB * DHW, C)


def _prep_body(xd_ref, xh_ref, xw_ref,
               bidx_ref, bw_ref, nidx_ref, nw_ref):
    xd = xd_ref[...]
    xh = xh_ref[...]
    xw = xw_ref[...]
    noff = lax.broadcasted_iota(jnp.int32, (NB, KP), 0) * DHW

    ix = ((xw + 1.0) * NWW - 1.0) / 2.0
    iy = ((xh + 1.0) * NHH - 1.0) / 2.0
    iz = ((xd + 1.0) * ND - 1.0) / 2.0
    x0 = jnp.floor(ix); y0 = jnp.floor(iy); z0 = jnp.floor(iz)
    wx1 = ix - x0; wy1 = iy - y0; wz1 = iz - z0
    wx0 = 1.0 - wx1; wy0 = 1.0 - wy1; wz0 = 1.0 - wz1
    x0i = x0.astype(jnp.int32); y0i = y0.astype(jnp.int32); z0i = z0.astype(jnp.int32)
    corner = 0
    for dz, wz in ((0, wz0), (1, wz1)):
        for dy, wy in ((0, wy0), (1, wy1)):
            for dx, wx in ((0, wx0), (1, wx1)):
                zi = z0i + dz; yi = y0i + dy; xi = x0i + dx
                mask = ((zi >= 0) & (zi < ND) & (yi >= 0) & (yi < NHH)
                        & (xi >= 0) & (xi < NWW))
                zc = jnp.clip(zi, 0, ND - 1)
                yc = jnp.clip(yi, 0, NHH - 1)
                xc = jnp.clip(xi, 0, NWW - 1)
                flat = (zc * NHH + yc) * NWW + xc + noff
                bidx_ref[corner] = flat
                bw_ref[corner] = (wz * wy * wx) * mask.astype(jnp.float32)
                corner += 1

    gd = jnp.floor((xd + 1.0) / 2.0 * (ND - 1.0))
    gh = jnp.floor((xh + 1.0) / 2.0 * (NHH - 1.0))
    gw = jnp.floor((xw + 1.0) / 2.0 * (NWW - 1.0))
    ncn_d = gd / (ND - 1) * 2 - 1
    rws = []
    masks = []
    flats = []
    for i in range(3):
        for j in range(3):
            dh = (i - 1) * (2.0 / NHH)
            dv = (j - 1) * (2.0 / NWW)
            ncn_h = (gh + dh) / (NHH - 1) * 2 - 1
            ncn_w = (gw + dv) / (NWW - 1) * 2 - 1
            sx = ((ncn_w + 1.0) * NWW - 1.0) / 2.0
            sy = ((ncn_h + 1.0) * NHH - 1.0) / 2.0
            sz = ((ncn_d + 1.0) * ND - 1.0) / 2.0
            xi = jnp.round(sx).astype(jnp.int32)
            yi = jnp.round(sy).astype(jnp.int32)
            zi = jnp.round(sz).astype(jnp.int32)
            mask = ((zi >= 0) & (zi < ND) & (yi >= 0) & (yi < NHH)
                    & (xi >= 0) & (xi < NWW))
            zc = jnp.clip(zi, 0, ND - 1)
            yc = jnp.clip(yi, 0, NHH - 1)
            xc = jnp.clip(xi, 0, NWW - 1)
            flat = (zc * NHH + yc) * NWW + xc + noff
            flats.append(jnp.where(mask, flat, NB * DHW))
            masks.append(mask)
            rd = jnp.sqrt((xd - ncn_d) ** 2 + (xh - ncn_h) ** 2
                          + (xw - ncn_w) ** 2)
            rws.append(1.0 / (rd + 1e-06))
    rwsum = rws[0]
    for a in range(1, 9):
        rwsum = rwsum + rws[a]
    for a in range(9):
        nidx_ref[a] = flats[a]
        nw_ref[a] = rws[a] / rwsum


def _prep(xyz_hr):
    xd = xyz_hr[:, :, 0]
    xh = xyz_hr[:, :, 1]
    xw = xyz_hr[:, :, 2]
    outs = pl.pallas_call(
        _prep_body,
        in_specs=[pl.BlockSpec((NB, KP), lambda: (0, 0))] * 3,
        out_specs=[
            pl.BlockSpec((8, NB, KP), lambda: (0, 0, 0)),
            pl.BlockSpec((8, NB, KP), lambda: (0, 0, 0)),
            pl.BlockSpec((9, NB, KP), lambda: (0, 0, 0)),
            pl.BlockSpec((9, NB, KP), lambda: (0, 0, 0)),
        ],
        out_shape=[
            jax.ShapeDtypeStruct((8, NB, KP), jnp.int32),
            jax.ShapeDtypeStruct((8, NB, KP), jnp.float32),
            jax.ShapeDtypeStruct((9, NB, KP), jnp.int32),
            jax.ShapeDtypeStruct((9, NB, KP), jnp.float32),
        ],
    )(xd, xh, xw)
    bidx, bw, nidx, nw = outs
    bidx = jnp.concatenate([jnp.transpose(bidx, (1, 2, 0)).reshape(P * 8),
                            jnp.zeros((G * 8,), jnp.int32)])
    bw = jnp.concatenate([jnp.transpose(bw, (1, 2, 0)).reshape(P * 8),
                          jnp.zeros((G * 8,), jnp.float32)])
    nidx = jnp.transpose(nidx, (1, 2, 0)).reshape(P * 9)
    rw_flat = jnp.transpose(nw, (1, 2, 0)).reshape(NB, KP * 9)
    rw_t = (jnp.transpose(rw_flat.reshape(NB, C, 576), (0, 2, 1)) * 0.5
            ).reshape(NB * 576 * C)
    return bidx, bw, nidx, rw_t


NC_SC = 2
NS_SC = 16
NWK = NC_SC * NS_SC
PTS_W = P // NWK
G = 8
BITERS = PTS_W // G
NPB = 576
UNITS = NB * 64
UNITS_W = UNITS // NWK
NCH = C // 16


def _gather_units(fm_table, sim_table, nidx, rw_t):
    mesh = plsc.VectorSubcoreMesh(core_axis_name="c", subcore_axis_name="s")

    @functools.partial(
        pl.kernel, mesh=mesh,
        out_type=jax.ShapeDtypeStruct((UNITS * C, C), jnp.float32),
        scratch_types=[
            pltpu.VMEM((C,), jnp.int32),
            pltpu.VMEM((C,), jnp.int32),
            pltpu.VMEM((C,), jnp.float32),
            pltpu.VMEM((C,), jnp.float32),
            pltpu.VMEM((C, C), jnp.float32),
            pltpu.VMEM((C, C), jnp.float32),
            pltpu.VMEM((C, C), jnp.float32),
            pltpu.VMEM((C, C), jnp.float32),
            pltpu.VMEM((C, C), jnp.float32),
            pltpu.SemaphoreType.DMA,
            pltpu.SemaphoreType.DMA,
        ],
    )
    def sc_kernel(fm_hbm, sim_hbm, nidx_hbm, rw_hbm,
                  u_hbm,
                  nidx0, nidx1, wrow0, wrow1,
                  rows_fm0, rows_fm1, rows_sim0, rows_sim1, u_acc,
                  sem0, sem1):
        wid = lax.axis_index("s") * NC_SC + lax.axis_index("c")
        nidx_b = (nidx0, nidx1)
        wrow_b = (wrow0, wrow1)
        rfm_b = (rows_fm0, rows_fm1)
        rsim_b = (rows_sim0, rows_sim1)
        sem_b = (sem0, sem1)

        def fetch_sub(nn, qq, a, p):
            sbase = pl.multiple_of(nn * (KP * 9) + qq * 1152 + a * C, C)
            wbase = pl.multiple_of((nn * NPB + qq * 9 + a) * C, C)
            pltpu.sync_copy(nidx_hbm.at[pl.ds(sbase, C)], nidx_b[p])
            pltpu.sync_copy(rw_hbm.at[pl.ds(wbase, C)], wrow_b[p])
            c1 = pltpu.async_copy(fm_hbm.at[nidx_b[p]], rfm_b[p], sem_b[p])
            c2 = pltpu.async_copy(sim_hbm.at[nidx_b[p]], rsim_b[p], sem_b[p])
            return c1, c2

        def unit_body(t, _):
            unit = wid * UNITS_W + t
            nn = unit // 64
            qq = unit - nn * 64

            def zero_row(r, _z):
                for c8 in range(NCH):
                    u_acc[r, pl.ds(c8 * 16, 16)] = jnp.zeros((16,), jnp.float32)
                return _z
            lax.fori_loop(0, C // 2, lambda r, z: zero_row(2 * r, zero_row(2 * r + 1, z)), 0)

            cps = fetch_sub(nn, qq, 0, 0)
            for a in range(9):
                p = a % 2
                nxt = cps
                if a + 1 < 9:
                    cps = fetch_sub(nn, qq, a + 1, 1 - p)
                nxt[0].wait()
                nxt[1].wait()
                wchunks = [wrow_b[p][pl.ds(c8 * 16, 16)] for c8 in range(NCH)]
                rfm = rfm_b[p]
                rsim = rsim_b[p]

                def row_body(r2, _r, rfm=rfm, rsim=rsim, wchunks=wchunks):
                    for u in range(2):
                        r = r2 * 2 + u
                        for c8 in range(NCH):
                            sl = pl.ds(c8 * 16, 16)
                            val = (rfm[r, sl] + rsim[r, sl]) * wchunks[c8]
                            plsc.addupdate(u_acc.at[r, sl], val)
                    return _r
                lax.fori_loop(0, C // 2, row_body, 0)
            ub = pl.multiple_of(unit * C, C)
            pltpu.sync_copy(u_acc, u_hbm.at[pl.ds(ub, C)])
            return _
        lax.fori_loop(0, UNITS_W, unit_body, 0)

    return sc_kernel(fm_table, sim_table, nidx, rw_t)


def _gather_bilinear(fm_table, bidx, bw):
    mesh = plsc.VectorSubcoreMesh(core_axis_name="c", subcore_axis_name="s")

    @functools.partial(
        pl.kernel, mesh=mesh,
        out_type=jax.ShapeDtypeStruct((P, C), jnp.float32),
        scratch_types=[
            pltpu.VMEM((G * 8,), jnp.int32),
            pltpu.VMEM((G * 8,), jnp.int32),
            pltpu.VMEM((G * 8,), jnp.float32),
            pltpu.VMEM((G * 8,), jnp.float32),
            pltpu.VMEM((G * 8, C), jnp.float32),
            pltpu.VMEM((G * 8, C), jnp.float32),
            pltpu.VMEM((G, C), jnp.float32),
            pltpu.SemaphoreType.DMA,
            pltpu.SemaphoreType.DMA,
        ],
    )
    def sc_kernel(fm_hbm, bidx_hbm, bw_hbm,
                  init_hbm,
                  bidx0, bidx1, bw0, bw1, brows0, brows1, out_i,
                  sem0, sem1):
        wid = lax.axis_index("s") * NC_SC + lax.axis_index("c")
        sem_b = (sem0, sem1)
        wbase_pts = wid * PTS_W
        bidx_d = (bidx0, bidx1)
        bw_d = (bw0, bw1)
        brows_d = (brows0, brows1)

        def fetch_bil(chunk, p):
            b8 = pl.multiple_of(chunk * (G * 8), G * 8)
            pltpu.sync_copy(bidx_hbm.at[pl.ds(b8, G * 8)], bidx_d[p])
            pltpu.sync_copy(bw_hbm.at[pl.ds(b8, G * 8)], bw_d[p])
            pltpu.async_copy(fm_hbm.at[bidx_d[p]], brows_d[p], sem_b[p])

        fetch_bil(wid * BITERS, 0)

        def bil2_body(i2, _):
            for b in range(2):
                chunk = i2 * 2 + b
                base = wbase_pts + chunk * G
                ob = pl.multiple_of(base, G)
                fetch_bil(wid * BITERS + chunk + 1, 1 - b)
                pltpu.make_async_copy(fm_hbm.at[bidx_d[b]],
                                      brows_d[b], sem_b[b]).wait()
                bwv = bw_d[b]
                brr = brows_d[b]

                def pt_body(pair, _p, bwv=bwv, brr=brr):
                    wv = bwv[pl.ds(pair * 16, 16)]
                    for half in range(2):
                        g = pair * 2 + half
                        wb = [wv[half * 8 + j] for j in range(8)]
                        for c8 in range(NCH):
                            sl = pl.ds(c8 * 16, 16)
                            acc = wb[0] * brr[g * 8, sl]
                            for j in range(1, 8):
                                acc = acc + wb[j] * brr[g * 8 + j, sl]
                            out_i[g, sl] = acc
                    return _p
                lax.fori_loop(0, G // 2, pt_body, 0)
                pltpu.sync_copy(out_i, init_hbm.at[pl.ds(ob, G)])
            return _
        lax.fori_loop(0, BITERS // 2, bil2_body, 0)
        pltpu.make_async_copy(fm_hbm.at[bidx_d[0]], brows_d[0], sem_b[0]).wait()

    return sc_kernel(fm_table, bidx, bw)


def _attn_body(init_ref, comb_ref, wq_ref, bq_ref, wk_ref, bk_ref,
               wv_ref, bv_ref, wqi_ref, wki_ref, wvi_ref, bi_ref,
               wo_ref, bo_ref, out_ref):
    def aff(x, w_ref, b=None):
        y = lax.dot_general(x, w_ref[...], (((1,), (1,)), ((), ())),
                            preferred_element_type=jnp.float32)
        if b is not None:
            y = y + b[...]
        return y

    i_l = [init_ref[0], init_ref[1]]
    c_l = [comb_ref[0], comb_ref[1]]
    q = [aff(i_l[l], wq_ref, bq_ref) for l in range(2)]
    k = [aff(c_l[l], wk_ref, bk_ref) for l in range(2)]
    v = [aff(c_l[l], wv_ref, bv_ref) for l in range(2)]
    bi = bi_ref[...]
    qp = [aff(q[l], wqi_ref) + bi[0:1] for l in range(2)]
    kp = [aff(k[l], wki_ref) + bi[1:2] for l in range(2)]
    vp = [aff(v[l], wvi_ref) + bi[2:3] for l in range(2)]

    seg = (lax.broadcasted_iota(jnp.int32, (C, NH), 0) // DH
           == lax.broadcasted_iota(jnp.int32, (C, NH), 1)).astype(jnp.float32)
    segT = (lax.broadcasted_iota(jnp.int32, (NH, C), 0)
            == lax.broadcasted_iota(jnp.int32, (NH, C), 1) // DH).astype(jnp.float32)
    scale = 1.0 / (DH ** 0.5)

    for l in range(2):
        s0 = lax.dot_general(qp[l] * kp[0], seg, (((1,), (0,)), ((), ())),
                             preferred_element_type=jnp.float32) * scale
        s1 = lax.dot_general(qp[l] * kp[1], seg, (((1,), (0,)), ((), ())),
                             preferred_element_type=jnp.float32) * scale
        m = jnp.maximum(s0, s1)
        e0 = jnp.exp(s0 - m)
        e1 = jnp.exp(s1 - m)
        den = e0 + e1
        a0 = lax.dot_general(e0 / den, segT, (((1,), (0,)), ((), ())),
                             preferred_element_type=jnp.float32)
        a1 = lax.dot_general(e1 / den, segT, (((1,), (0,)), ((), ())),
                             preferred_element_type=jnp.float32)
        o = a0 * vp[0] + a1 * vp[1]
        out_ref[l] = aff(o, wo_ref, bo_ref) + i_l[l]


def _attention(init_fv, combined, Wq, bq, Wk, bk, Wv, bv,
               in_proj_w, in_proj_b, out_proj_w, out_proj_b):
    TD = 2048
    wqi = in_proj_w[0:C]
    wki = in_proj_w[C:2 * C]
    wvi = in_proj_w[2 * C:3 * C]
    bi = in_proj_b.reshape(3, C)
    full = pl.BlockSpec((C, C), lambda t: (0, 0))
    bias = pl.BlockSpec((1, C), lambda t: (0, 0))
    return pl.pallas_call(
        _attn_body,
        grid=(KP // TD,),
        in_specs=[
            pl.BlockSpec((NB, TD, C), lambda t: (0, t, 0)),
            pl.BlockSpec((NB, TD, C), lambda t: (0, t, 0)),
            full, bias, full, bias, full, bias,
            full, full, full, pl.BlockSpec((3, C), lambda t: (0, 0)),
            full, bias,
        ],
        out_specs=pl.BlockSpec((NB, TD, C), lambda t: (0, t, 0)),
        out_shape=jax.ShapeDtypeStruct((NB, KP, C), jnp.float32),
    )(init_fv, combined, Wq, bq.reshape(1, C), Wk, bk.reshape(1, C),
      Wv, bv.reshape(1, C), wqi, wki, wvi, bi, out_proj_w,
      out_proj_b.reshape(1, C))


def kernel(feature_map, xyz_hr, Wq, bq, Wk, bk, Wv, bv,
           in_proj_w, in_proj_b, out_proj_w, out_proj_b):
    zrow = jnp.zeros((1, C), jnp.float32)
    fm_table = jnp.concatenate(
        [jnp.transpose(feature_map, (0, 2, 3, 4, 1)).reshape(NB * DHW, C), zrow])
    bidx, bw, nidx, rw_t = _prep(xyz_hr)
    init_flat = _gather_bilinear(fm_table, bidx, bw)
    sim_table = jnp.concatenate([_similar_table(feature_map), zrow])
    u_flat = _gather_units(fm_table, sim_table, nidx, rw_t)
    init_fv = init_flat.reshape(NB, KP, C)
    combined = jnp.transpose(u_flat.reshape(NB, 64, C, C),
                             (0, 3, 1, 2)).reshape(NB, KP, C)
    return _attention(init_fv, combined, Wq, bq, Wk, bk, Wv, bv,
                      in_proj_w, in_proj_b, out_proj_w, out_proj_b)


# speedup vs baseline: 27.4729x; 1.0016x over previous
import functools

import jax
import jax.numpy as jnp
from jax import lax
from jax.experimental import pallas as pl
from jax.experimental.pallas import tpu as pltpu
from jax.experimental.pallas import tpu_sc as plsc

C = 128
NH = 8
DH = C // NH
TOPK = 5
ND, NHH, NWW = 16, 32, 32
LHW = NHH * NWW
NB = 2
KP = 8192
P = NB * KP
DHW = ND * LHW


RB = LHW
NRB = LHW // RB


def _sim_body(sf_ref, out_ref):
    sf = sf_ref[0, 0]
    r0 = pl.program_id(2) * RB
    sfr = sf_ref[0, 0, :, pl.ds(r0, RB)]
    sim = lax.dot_general(sfr, sf, (((0,), (0,)), ((), ())),
                          preferred_element_type=jnp.float32)
    icol = lax.broadcasted_iota(jnp.int32, (RB, LHW), 1)
    irow = lax.broadcasted_iota(jnp.int32, (RB, 1), 0) + r0
    wmat_raw = jnp.zeros((RB, LHW), jnp.float32)
    wraws = []
    for _ in range(TOPK):
        m = jnp.max(sim, axis=1, keepdims=True)
        cand = jnp.where(sim == m, icol, LHW)
        idx = jnp.min(cand, axis=1, keepdims=True)
        wr = 1.0 / (jnp.abs(idx - irow).astype(jnp.float32) + 1e-05)
        e = icol == idx
        sim = jnp.where(e, -jnp.inf, sim)
        wmat_raw = jnp.where(e, wr, wmat_raw)
        wraws.append(wr)
    rinv = 1.0 / (wraws[0] + wraws[1] + wraws[2] + wraws[3] + wraws[4])
    out_ref[0, 0] = lax.dot_general(wmat_raw * rinv, sf,
                                    (((1,), (1,)), ((), ())),
                                    preferred_element_type=jnp.float32)


def _similar_table(feature_map):
    slices = jnp.transpose(feature_map, (0, 2, 1, 3, 4)).reshape(NB, ND, C, LHW)
    out = pl.pallas_call(
        _sim_body,
        grid=(NB, ND, NRB),
        in_specs=[pl.BlockSpec((1, 1, C, LHW), lambda n, d, r: (n, d, 0, 0))],
        out_specs=pl.BlockSpec((1, 1, RB, C), lambda n, d, r: (d, n, r, 0)),
        out_shape=jax.ShapeDtypeStruct((ND, NB, LHW, C), jnp.float32),
    )(slices)
    weighted = jnp.transpose(out, (0, 1, 3, 2))
    sim_vol = weighted.reshape(NB, C, ND, LHW)
    return jnp.transpose(sim_vol, (0, 2, 3, 1)).reshape(NB * DHW, C)


def _prep_body(xd_ref, xh_ref, xw_ref,
               bidx_ref, bw_ref, nidx_ref, nw_ref):
    xd = xd_ref[...]
    xh = xh_ref[...]
    xw = xw_ref[...]
    noff = lax.broadcasted_iota(jnp.int32, (NB, KP), 0) * DHW

    ix = ((xw + 1.0) * NWW - 1.0) / 2.0
    iy = ((xh + 1.0) * NHH - 1.0) / 2.0
    iz = ((xd + 1.0) * ND - 1.0) / 2.0
    x0 = jnp.floor(ix); y0 = jnp.floor(iy); z0 = jnp.floor(iz)
    wx1 = ix - x0; wy1 = iy - y0; wz1 = iz - z0
    wx0 = 1.0 - wx1; wy0 = 1.0 - wy1; wz0 = 1.0 - wz1
    x0i = x0.astype(jnp.int32); y0i = y0.astype(jnp.int32); z0i = z0.astype(jnp.int32)
    corner = 0
    for dz, wz in ((0, wz0), (1, wz1)):
        for dy, wy in ((0, wy0), (1, wy1)):
            for dx, wx in ((0, wx0), (1, wx1)):
                zi = z0i + dz; yi = y0i + dy; xi = x0i + dx
                mask = ((zi >= 0) & (zi < ND) & (yi >= 0) & (yi < NHH)
                        & (xi >= 0) & (xi < NWW))
                zc = jnp.clip(zi, 0, ND - 1)
                yc = jnp.clip(yi, 0, NHH - 1)
                xc = jnp.clip(xi, 0, NWW - 1)
                flat = (zc * NHH + yc) * NWW + xc + noff
                bidx_ref[corner] = flat
                bw_ref[corner] = (wz * wy * wx) * mask.astype(jnp.float32)
                corner += 1

    gd = jnp.floor((xd + 1.0) / 2.0 * (ND - 1.0))
    gh = jnp.floor((xh + 1.0) / 2.0 * (NHH - 1.0))
    gw = jnp.floor((xw + 1.0) / 2.0 * (NWW - 1.0))
    ncn_d = gd / (ND - 1) * 2 - 1
    rws = []
    masks = []
    flats = []
    for i in range(3):
        for j in range(3):
            dh = (i - 1) * (2.0 / NHH)
            dv = (j - 1) * (2.0 / NWW)
            ncn_h = (gh + dh) / (NHH - 1) * 2 - 1
            ncn_w = (gw + dv) / (NWW - 1) * 2 - 1
            sx = ((ncn_w + 1.0) * NWW - 1.0) / 2.0
            sy = ((ncn_h + 1.0) * NHH - 1.0) / 2.0
            sz = ((ncn_d + 1.0) * ND - 1.0) / 2.0
            xi = jnp.round(sx).astype(jnp.int32)
            yi = jnp.round(sy).astype(jnp.int32)
            zi = jnp.round(sz).astype(jnp.int32)
            mask = ((zi >= 0) & (zi < ND) & (yi >= 0) & (yi < NHH)
                    & (xi >= 0) & (xi < NWW))
            zc = jnp.clip(zi, 0, ND - 1)
            yc = jnp.clip(yi, 0, NHH - 1)
            xc = jnp.clip(xi, 0, NWW - 1)
            flat = (zc * NHH + yc) * NWW + xc + noff
            flats.append(jnp.where(mask, flat, NB * DHW))
            masks.append(mask)
            rd = jnp.sqrt((xd - ncn_d) ** 2 + (xh - ncn_h) ** 2
                          + (xw - ncn_w) ** 2)
            rws.append(1.0 / (rd + 1e-06))
    rwsum = rws[0]
    for a in range(1, 9):
        rwsum = rwsum + rws[a]
    for a in range(9):
        nidx_ref[a] = flats[a]
        nw_ref[a] = rws[a] / rwsum


def _prep(xyz_hr):
    xd = xyz_hr[:, :, 0]
    xh = xyz_hr[:, :, 1]
    xw = xyz_hr[:, :, 2]
    outs = pl.pallas_call(
        _prep_body,
        in_specs=[pl.BlockSpec((NB, KP), lambda: (0, 0))] * 3,
        out_specs=[
            pl.BlockSpec((8, NB, KP), lambda: (0, 0, 0)),
            pl.BlockSpec((8, NB, KP), lambda: (0, 0, 0)),
            pl.BlockSpec((9, NB, KP), lambda: (0, 0, 0)),
            pl.BlockSpec((9, NB, KP), lambda: (0, 0, 0)),
        ],
        out_shape=[
            jax.ShapeDtypeStruct((8, NB, KP), jnp.int32),
            jax.ShapeDtypeStruct((8, NB, KP), jnp.float32),
            jax.ShapeDtypeStruct((9, NB, KP), jnp.int32),
            jax.ShapeDtypeStruct((9, NB, KP), jnp.float32),
        ],
    )(xd, xh, xw)
    bidx, bw, nidx, nw = outs
    bidx = jnp.concatenate([jnp.transpose(bidx, (1, 2, 0)).reshape(P * 8),
                            jnp.zeros((G * 8,), jnp.int32)])
    bw = jnp.concatenate([jnp.transpose(bw, (1, 2, 0)).reshape(P * 8),
                          jnp.zeros((G * 8,), jnp.float32)])
    nidx = jnp.transpose(nidx, (1, 2, 0)).reshape(P * 9)
    rw_flat = jnp.transpose(nw, (1, 2, 0)).reshape(NB, KP * 9)
    rw_t = (jnp.transpose(rw_flat.reshape(NB, C, 576), (0, 2, 1)) * 0.5
            ).reshape(NB * 576 * C)
    return bidx, bw, nidx, rw_t


NC_SC = 2
NS_SC = 16
NWK = NC_SC * NS_SC
PTS_W = P // NWK
G = 8
BITERS = PTS_W // G
NPB = 576
UNITS = NB * 64
UNITS_W = UNITS // NWK
NCH = C // 16


def _gather_units(fm_table, sim_table, nidx, rw_t):
    mesh = plsc.VectorSubcoreMesh(core_axis_name="c", subcore_axis_name="s")

    @functools.partial(
        pl.kernel, mesh=mesh,
        out_type=jax.ShapeDtypeStruct((UNITS * C, C), jnp.float32),
        scratch_types=[
            pltpu.VMEM((C,), jnp.int32),
            pltpu.VMEM((C,), jnp.int32),
            pltpu.VMEM((C,), jnp.float32),
            pltpu.VMEM((C,), jnp.float32),
            pltpu.VMEM((C, C), jnp.float32),
            pltpu.VMEM((C, C), jnp.float32),
            pltpu.VMEM((C, C), jnp.float32),
            pltpu.VMEM((C, C), jnp.float32),
            pltpu.VMEM((C, C), jnp.float32),
            pltpu.SemaphoreType.DMA,
            pltpu.SemaphoreType.DMA,
        ],
    )
    def sc_kernel(fm_hbm, sim_hbm, nidx_hbm, rw_hbm,
                  u_hbm,
                  nidx0, nidx1, wrow0, wrow1,
                  rows_fm0, rows_fm1, rows_sim0, rows_sim1, u_acc,
                  sem0, sem1):
        wid = lax.axis_index("s") * NC_SC + lax.axis_index("c")
        nidx_b = (nidx0, nidx1)
        wrow_b = (wrow0, wrow1)
        rfm_b = (rows_fm0, rows_fm1)
        rsim_b = (rows_sim0, rows_sim1)
        sem_b = (sem0, sem1)

        def fetch_sub(nn, qq, a, p):
            sbase = pl.multiple_of(nn * (KP * 9) + qq * 1152 + a * C, C)
            wbase = pl.multiple_of((nn * NPB + qq * 9 + a) * C, C)
            pltpu.sync_copy(nidx_hbm.at[pl.ds(sbase, C)], nidx_b[p])
            pltpu.sync_copy(rw_hbm.at[pl.ds(wbase, C)], wrow_b[p])
            c1 = pltpu.async_copy(fm_hbm.at[nidx_b[p]], rfm_b[p], sem_b[p])
            c2 = pltpu.async_copy(sim_hbm.at[nidx_b[p]], rsim_b[p], sem_b[p])
            return c1, c2

        def unit_body(t, _):
            unit = wid * UNITS_W + t
            nn = unit // 64
            qq = unit - nn * 64

            def zero_row(r, _z):
                for c8 in range(NCH):
                    u_acc[r, pl.ds(c8 * 16, 16)] = jnp.zeros((16,), jnp.float32)
                return _z
            lax.fori_loop(0, C // 2, lambda r, z: zero_row(2 * r, zero_row(2 * r + 1, z)), 0)

            cps = fetch_sub(nn, qq, 0, 0)
            for a in range(9):
                p = a % 2
                nxt = cps
                if a + 1 < 9:
                    cps = fetch_sub(nn, qq, a + 1, 1 - p)
                nxt[0].wait()
                nxt[1].wait()
                wchunks = [wrow_b[p][pl.ds(c8 * 16, 16)] for c8 in range(NCH)]
                rfm = rfm_b[p]
                rsim = rsim_b[p]

                def row_body(r2, _r, rfm=rfm, rsim=rsim, wchunks=wchunks):
                    for u in range(2):
                        r = r2 * 2 + u
                        for c8 in range(NCH):
                            sl = pl.ds(c8 * 16, 16)
                            val = (rfm[r, sl] + rsim[r, sl]) * wchunks[c8]
                            plsc.addupdate(u_acc.at[r, sl], val)
                    return _r
                lax.fori_loop(0, C // 2, row_body, 0)
            ub = pl.multiple_of(unit * C, C)
            pltpu.sync_copy(u_acc, u_hbm.at[pl.ds(ub, C)])
            return _
        lax.fori_loop(0, UNITS_W, unit_body, 0)

    return sc_kernel(fm_table, sim_table, nidx, rw_t)


def _gather_bilinear(fm_table, bidx, bw):
    mesh = plsc.VectorSubcoreMesh(core_axis_name="c", subcore_axis_name="s")

    @functools.partial(
        pl.kernel, mesh=mesh,
        out_type=jax.ShapeDtypeStruct((P, C), jnp.float32),
        scratch_types=[
            pltpu.VMEM((G * 8,), jnp.int32),
            pltpu.VMEM((G * 8,), jnp.int32),
            pltpu.VMEM((G * 8,), jnp.float32),
            pltpu.VMEM((G * 8,), jnp.float32),
            pltpu.VMEM((G * 8, C), jnp.float32),
            pltpu.VMEM((G * 8, C), jnp.float32),
            pltpu.VMEM((G, C), jnp.float32),
            pltpu.SemaphoreType.DMA,
            pltpu.SemaphoreType.DMA,
        ],
    )
    def sc_kernel(fm_hbm, bidx_hbm, bw_hbm,
                  init_hbm,
                  bidx0, bidx1, bw0, bw1, brows0, brows1, out_i,
                  sem0, sem1):
        wid = lax.axis_index("s") * NC_SC + lax.axis_index("c")
        sem_b = (sem0, sem1)
        wbase_pts = wid * PTS_W
        bidx_d = (bidx0, bidx1)
        bw_d = (bw0, bw1)
        brows_d = (brows0, brows1)

        def fetch_bil(chunk, p):
            b8 = pl.multiple_of(chunk * (G * 8), G * 8)
            pltpu.sync_copy(bidx_hbm.at[pl.ds(b8, G * 8)], bidx_d[p])
            pltpu.sync_copy(bw_hbm.at[pl.ds(b8, G * 8)], bw_d[p])
            pltpu.async_copy(fm_hbm.at[bidx_d[p]], brows_d[p], sem_b[p])

        fetch_bil(wid * BITERS, 0)

        def bil2_body(i2, _):
            for b in range(2):
                chunk = i2 * 2 + b
                base = wbase_pts + chunk * G
                ob = pl.multiple_of(base, G)
                fetch_bil(wid * BITERS + chunk + 1, 1 - b)
                pltpu.make_async_copy(fm_hbm.at[bidx_d[b]],
                                      brows_d[b], sem_b[b]).wait()
                bwv = bw_d[b]
                brr = brows_d[b]

                def pt_body(pair, _p, bwv=bwv, brr=brr):
                    wv = bwv[pl.ds(pair * 16, 16)]
                    for half in range(2):
                        g = pair * 2 + half
                        wb = [wv[half * 8 + j] for j in range(8)]
                        for c8 in range(NCH):
                            sl = pl.ds(c8 * 16, 16)
                            acc = wb[0] * brr[g * 8, sl]
                            for j in range(1, 8):
                                acc = acc + wb[j] * brr[g * 8 + j, sl]
                            out_i[g, sl] = acc
                    return _p
                lax.fori_loop(0, G // 2, pt_body, 0)
                pltpu.sync_copy(out_i, init_hbm.at[pl.ds(ob, G)])
            return _
        lax.fori_loop(0, BITERS // 2, bil2_body, 0)
        pltpu.make_async_copy(fm_hbm.at[bidx_d[0]], brows_d[0], sem_b[0]).wait()

    return sc_kernel(fm_table, bidx, bw)


def _attn_body(init_ref, comb_ref, wq_ref, bq_ref, wk_ref, bk_ref,
               wv_ref, bv_ref, wqi_ref, wki_ref, wvi_ref, bi_ref,
               wo_ref, bo_ref, out_ref):
    def aff(x, w_ref, b=None):
        y = lax.dot_general(x, w_ref[...], (((1,), (1,)), ((), ())),
                            preferred_element_type=jnp.float32)
        if b is not None:
            y = y + b[...]
        return y

    i_l = [init_ref[0], init_ref[1]]
    c_l = [comb_ref[0], comb_ref[1]]
    q = [aff(i_l[l], wq_ref, bq_ref) for l in range(2)]
    k = [aff(c_l[l], wk_ref, bk_ref) for l in range(2)]
    v = [aff(c_l[l], wv_ref, bv_ref) for l in range(2)]
    bi = bi_ref[...]
    qp = [aff(q[l], wqi_ref) + bi[0:1] for l in range(2)]
    kp = [aff(k[l], wki_ref) + bi[1:2] for l in range(2)]
    vp = [aff(v[l], wvi_ref) + bi[2:3] for l in range(2)]

    seg = (lax.broadcasted_iota(jnp.int32, (C, NH), 0) // DH
           == lax.broadcasted_iota(jnp.int32, (C, NH), 1)).astype(jnp.float32)
    segT = (lax.broadcasted_iota(jnp.int32, (NH, C), 0)
            == lax.broadcasted_iota(jnp.int32, (NH, C), 1) // DH).astype(jnp.float32)
    scale = 1.0 / (DH ** 0.5)

    for l in range(2):
        s0 = lax.dot_general(qp[l] * kp[0], seg, (((1,), (0,)), ((), ())),
                             preferred_element_type=jnp.float32) * scale
        s1 = lax.dot_general(qp[l] * kp[1], seg, (((1,), (0,)), ((), ())),
                             preferred_element_type=jnp.float32) * scale
        m = jnp.maximum(s0, s1)
        e0 = jnp.exp(s0 - m)
        e1 = jnp.exp(s1 - m)
        den = e0 + e1
        a0 = lax.dot_general(e0 / den, segT, (((1,), (0,)), ((), ())),
                             preferred_element_type=jnp.float32)
        a1 = lax.dot_general(e1 / den, segT, (((1,), (0,)), ((), ())),
                             preferred_element_type=jnp.float32)
        o = a0 * vp[0] + a1 * vp[1]
        out_ref[l] = aff(o, wo_ref, bo_ref) + i_l[l]


def _attention(init_fv, combined, Wq, bq, Wk, bk, Wv, bv,
               in_proj_w, in_proj_b, out_proj_w, out_proj_b):
    TD = 2048
    wqi = in_proj_w[0:C]
    wki = in_proj_w[C:2 * C]
    wvi = in_proj_w[2 * C:3 * C]
    bi = in_proj_b.reshape(3, C)
    full = pl.BlockSpec((C, C), lambda t: (0, 0))
    bias = pl.BlockSpec((1, C), lambda t: (0, 0))
    return pl.pallas_call(
        _attn_body,
        grid=(KP // TD,),
        in_specs=[
            pl.BlockSpec((NB, TD, C), lambda t: (0, t, 0)),
            pl.BlockSpec((NB, TD, C), lambda t: (0, t, 0)),
            full, bias, full, bias, full, bias,
            full, full, full, pl.BlockSpec((3, C), lambda t: (0, 0)),
            full, bias,
        ],
        out_specs=pl.BlockSpec((NB, TD, C), lambda t: (0, t, 0)),
        out_shape=jax.ShapeDtypeStruct((NB, KP, C), jnp.float32),
    )(init_fv, combined, Wq, bq.reshape(1, C), Wk, bk.reshape(1, C),
      Wv, bv.reshape(1, C), wqi, wki, wvi, bi, out_proj_w,
      out_proj_b.reshape(1, C))


def kernel(feature_map, xyz_hr, Wq, bq, Wk, bk, Wv, bv,
           in_proj_w, in_proj_b, out_proj_w, out_proj_b):
    zrow = jnp.zeros((1, C), jnp.float32)
    fm_table = jnp.concatenate(
        [jnp.transpose(feature_map, (0, 2, 3, 4, 1)).reshape(NB * DHW, C), zrow])
    bidx, bw, nidx, rw_t = _prep(xyz_hr)
    init_flat = _gather_bilinear(fm_table, bidx, bw)
    sim_table = jnp.concatenate([_similar_table(feature_map), zrow])
    u_flat = _gather_units(fm_table, sim_table, nidx, rw_t)
    init_fv = init_flat.reshape(NB, KP, C)
    combined = jnp.transpose(u_flat.reshape(NB, 64, C, C),
                             (0, 3, 1, 2)).reshape(NB, KP, C)
    return _attention(init_fv, combined, Wq, bq, Wk, bk, Wv, bv,
                      in_proj_w, in_proj_b, out_proj_w, out_proj_b)
